# Initial kernel scaffold; baseline (speedup 1.0000x reference)
import jax
import jax.numpy as jnp
from jax.experimental import pallas as pl


def kernel(intensities, learnable_opacity, base_opacity_lut, color_lut, active_indices):
    raise NotImplementedError("write your pallas kernel here")


# speedup vs baseline: 38.2083x; 38.2083x over previous
import functools

import jax
import jax.numpy as jnp
from jax import lax
from jax.experimental import pallas as pl
from jax.experimental.pallas import tpu as pltpu
from jax.experimental.pallas import tpu_sc as plsc

_NUM_BINS = 256
_L = 16
_NC = 2
_NS = 16
_CHUNK = 4096


@functools.lru_cache(maxsize=None)
def _build_sc_kernel(n, na):
    nw = _NC * _NS
    per_w = n // nw
    n_chunks = per_w // _CHUNK
    assert per_w * nw == n and n_chunks * _CHUNK == per_w
    groups = _CHUNK // _L

    mesh = plsc.VectorSubcoreMesh(core_axis_name="c", subcore_axis_name="s")
    na_pad = ((na + _L - 1) // _L) * _L

    @functools.partial(
        pl.kernel,
        out_type=jax.ShapeDtypeStruct((n * 4,), jnp.float32),
        mesh=mesh,
        compiler_params=pltpu.CompilerParams(needs_layout_passes=False),
        scratch_types=[
            pltpu.VMEM((_NUM_BINS,), jnp.float32),
            pltpu.VMEM((_NUM_BINS, 3), jnp.float32),
            pltpu.VMEM((na_pad,), jnp.int32),
            pltpu.VMEM((na_pad,), jnp.float32),
            pltpu.VMEM((8 * _NUM_BINS,), jnp.float32),
            pltpu.VMEM((_CHUNK,), jnp.float32),
            pltpu.VMEM((_CHUNK * 4,), jnp.float32),
        ],
    )
    def sc_kernel(x_hbm, lrn_hbm, base_hbm, col_hbm, act_hbm, out_hbm,
                  opac_v, cvec_v, actv_v, lrn_v, tab_v, in_v, out_v):
        iota = lax.iota(jnp.int32, _L)

        pltpu.sync_copy(base_hbm, opac_v)
        pltpu.sync_copy(col_hbm, cvec_v)
        pltpu.sync_copy(act_hbm, actv_v.at[pl.ds(0, na)])
        pltpu.sync_copy(lrn_hbm, lrn_v.at[pl.ds(0, na)])

        for i in range(0, na, _L):
            idxv = actv_v[pl.ds(i, _L)]
            valv = lrn_v[pl.ds(i, _L)]
            rem = na - i
            if rem >= _L:
                plsc.store_scatter(opac_v, [idxv], valv)
            else:
                plsc.store_scatter(opac_v, [idxv], valv, mask=iota < rem)

        for i in range(_NUM_BINS // _L):
            ilow = iota + (i * _L)
            ihigh = jnp.minimum(ilow + 1, _NUM_BINS - 1)
            o_lo = opac_v[pl.ds(i * _L, _L)]
            o_hi = plsc.load_gather(opac_v, [ihigh])
            for c in range(3):
                cc = jnp.full((_L,), c, jnp.int32)
                v_lo = plsc.load_gather(cvec_v, [ilow, cc])
                v_hi = plsc.load_gather(cvec_v, [ihigh, cc])
                tab_v[pl.ds(c * _NUM_BINS + i * _L, _L)] = v_lo
                tab_v[pl.ds((4 + c) * _NUM_BINS + i * _L, _L)] = v_hi - v_lo
            tab_v[pl.ds(3 * _NUM_BINS + i * _L, _L)] = o_lo
            tab_v[pl.ds(7 * _NUM_BINS + i * _L, _L)] = o_hi - o_lo

        wid = lax.axis_index("s") * _NC + lax.axis_index("c")
        base = wid * per_w
        oiota4 = iota * 4

        def group_body(j, carry):
            x = in_v[pl.ds(j * _L, _L)]
            f = jnp.clip(x * 255.0, 0.0, 255.0)
            low = f.astype(jnp.int32)
            t = f - low.astype(jnp.float32)
            r = plsc.load_gather(tab_v, [low])
            g = plsc.load_gather(tab_v, [low + 256])
            b = plsc.load_gather(tab_v, [low + 512])
            o = plsc.load_gather(tab_v, [low + 768])
            dr = plsc.load_gather(tab_v, [low + 1024])
            dg = plsc.load_gather(tab_v, [low + 1280])
            db = plsc.load_gather(tab_v, [low + 1536])
            do = plsc.load_gather(tab_v, [low + 1792])
            obase = oiota4 + j * (4 * _L)
            plsc.store_scatter(out_v, [obase], r + t * dr)
            plsc.store_scatter(out_v, [obase + 1], g + t * dg)
            plsc.store_scatter(out_v, [obase + 2], b + t * db)
            plsc.store_scatter(out_v, [obase + 3], o + t * do)
            return carry

        def chunk_body(gidx, carry):
            off = base + gidx * _CHUNK
            pltpu.sync_copy(x_hbm.at[pl.ds(off, _CHUNK)], in_v)
            lax.fori_loop(0, groups, group_body, 0)
            pltpu.sync_copy(out_v, out_hbm.at[pl.ds(off * 4, _CHUNK * 4)])
            return carry

        lax.fori_loop(0, n_chunks, chunk_body, 0)

    return sc_kernel


def kernel(intensities, learnable_opacity, base_opacity_lut, color_lut,
           active_indices):
    n = intensities.shape[0]
    na = active_indices.shape[0]
    fn = _build_sc_kernel(n, na)
    out = fn(intensities, learnable_opacity, base_opacity_lut, color_lut,
             active_indices)
    return out.reshape(n, 4)
